# baseline (speedup 1.0000x reference)
import functools

import jax
import jax.numpy as jnp
from jax import lax
from jax.experimental import pallas as pl
from jax.experimental.pallas import tpu as pltpu
from jax.experimental.pallas import tpu_sc as plsc

VOC = 1000000
B = 16384
K = 20
EMB = 64

_NW = 32
_L = 16
_BPW = B // _NW
_CBLK = _BPW // 128
_NROWS = _BPW * K // 128
_Q = EMB // _L

_CH = 384
_NCH = 999936 // _CH
_TAIL = VOC - _NCH * _CH
_CPW = 82

_mesh = plsc.VectorSubcoreMesh(core_axis_name="c", subcore_axis_name="s")


@functools.partial(
    pl.kernel,
    out_type=[
        jax.ShapeDtypeStruct((VOC // 2, 128), jnp.float32),
        jax.ShapeDtypeStruct((VOC // 2, 128), jnp.float32),
    ],
    mesh=_mesh,
    compiler_params=pltpu.CompilerParams(needs_layout_passes=False),
    scratch_types=[
        pltpu.VMEM((EMB, _CH), jnp.float32),
        pltpu.VMEM((EMB, _CH), jnp.float32),
        pltpu.VMEM((_CH // 2, 128), jnp.float32),
        pltpu.VMEM((_CH // 2, 128), jnp.float32),
        pltpu.SemaphoreType.DMA,
        pltpu.SemaphoreType.DMA,
        pltpu.SemaphoreType.DMA,
        pltpu.SemaphoreType.DMA,
    ],
)
def _sc_convert(wct_hbm, wot_hbm, tails_hbm, outc_hbm, outo_hbm,
                vbufa, vbufb, obufa, obufb, rsa, rsb, wsa, wsb):
    wid = lax.axis_index("s") * 2 + lax.axis_index("c")

    iota = lax.iota(jnp.int32, _L)
    vbufs = (vbufa, vbufb)
    obufs = (obufa, obufb)
    rsems = (rsa, rsb)
    wsems = (wsa, wsb)

    def transpose_chunk(vbuf, obuf):
        rows = [lax.shift_right_logical(g * _L + iota, 1)
                for g in range(_CH // _L)]
        colbase = [((g * _L + iota) & 1) * EMB for g in range(_CH // _L)]

        @plsc.parallel_loop(0, EMB, unroll=4)
        def tr(e):
            for g in range(_CH // _L):
                x = vbuf[e, pl.ds(g * _L, _L)]
                plsc.store_scatter(obuf, [rows[g], colbase[g] + e], x)

    def convert(src_hbm, out_hbm):
        base = wid * _CPW

        def chunk_of(k):
            return jnp.minimum(base + k, _NCH - 1)

        def read_descs(k, pb):
            v0 = chunk_of(k) * _CH
            return [
                pltpu.make_async_copy(
                    src_hbm.at[pl.ds(8 * t, 8), pl.ds(v0, _CH)],
                    vbufs[pb].at[pl.ds(8 * t, 8)], rsems[pb])
                for t in range(EMB // 8)
            ]

        def write_desc(k, pb):
            return pltpu.make_async_copy(
                obufs[pb],
                out_hbm.at[pl.ds(chunk_of(k) * (_CH // 2), _CH // 2)],
                wsems[pb])

        for d in read_descs(0, 0):
            d.start()

        def pair_body(p, carry):
            for pb in range(2):
                k = p * 2 + pb
                for d in read_descs(k, pb):
                    d.wait()

                @pl.when(k + 1 < _CPW)
                def _():
                    for d in read_descs(k + 1, (pb + 1) % 2):
                        d.start()

                @pl.when(k >= 2)
                def _():
                    write_desc(k - 2, pb).wait()

                transpose_chunk(vbufs[pb], obufs[pb])
                write_desc(k, pb).start()
            return carry

        lax.fori_loop(0, _CPW // 2, pair_body, 0)
        write_desc(_CPW - 2, 0).wait()
        write_desc(_CPW - 1, 1).wait()

    convert(wct_hbm, outc_hbm)
    convert(wot_hbm, outo_hbm)

    @pl.when(wid == 0)
    def _():
        for t, out_hbm in ((0, outc_hbm), (1, outo_hbm)):
            pltpu.async_copy(tails_hbm.at[t], obufa.at[pl.ds(0, 32)],
                             rsa).wait()
            pltpu.async_copy(obufa.at[pl.ds(0, 32)],
                             out_hbm.at[pl.ds(_NCH * (_CH // 2), 32)],
                             wsa).wait()


@functools.partial(
    pl.kernel,
    out_type=jax.ShapeDtypeStruct((_NW * _L,), jnp.float32),
    mesh=_mesh,
    compiler_params=pltpu.CompilerParams(needs_layout_passes=False),
    scratch_types=[
        pltpu.VMEM((8, 128), jnp.int32),
        pltpu.VMEM((_NROWS, 128), jnp.int32),
        pltpu.VMEM((_BPW * K,), jnp.int32),
        pltpu.VMEM((_BPW,), jnp.int32),
        pltpu.VMEM((_BPW, 128), jnp.float32),
        pltpu.VMEM((128, 128), jnp.float32),
        pltpu.VMEM((128, 128), jnp.float32),
        pltpu.VMEM((128,), jnp.float32),
        pltpu.VMEM((_L,), jnp.float32),
        pltpu.SemaphoreType.DMA,
        pltpu.SemaphoreType.DMA,
    ],
)
def _sc_loss(gco_hbm, gn_hbm, pkn_hbm, pou_hbm, wc2_hbm, wo2_hbm, out_hbm,
             idx_v, gn_v, pkn_v, pou_v, c_all, bufa, bufb, dblk, acc_v,
             sema, semb):
    wid = lax.axis_index("s") * 2 + lax.axis_index("c")

    pltpu.sync_copy(gco_hbm.at[pl.ds(wid * 8, 8)], idx_v)
    pltpu.sync_copy(gn_hbm.at[pl.ds(wid * _NROWS, _NROWS)], gn_v)
    pltpu.sync_copy(pkn_hbm.at[pl.ds(wid * _BPW * K, _BPW * K)], pkn_v)
    pltpu.sync_copy(pou_hbm.at[pl.ds(wid * _BPW, _BPW)], pou_v)

    iota = lax.iota(jnp.int32, _L)
    m15 = iota == _L - 1
    acc_v[...] = jnp.zeros((_L,), jnp.float32)

    cps = [
        pltpu.async_copy(wc2_hbm.at[idx_v.at[blk]],
                         c_all.at[pl.ds(blk * 128, 128)], sema)
        for blk in range(_CBLK)
    ]
    for cp in cps:
        cp.wait()

    def accum_dblk():
        def rb(t, a):
            x = dblk[pl.ds(pl.multiple_of(t * _L, _L), _L)]
            e = jnp.exp(-jnp.abs(x))
            z = e / (e + 2.0)
            z2 = z * z
            l1p = z * (2.0 + z2 * (2.0 / 3.0 + z2 * (2.0 / 5.0 + z2 * (2.0 / 7.0))))
            return a + (jnp.minimum(x, 0.0) - l1p)

        acc_v[...] = lax.fori_loop(0, 128 // _L, rb, acc_v[...])

    bufs = (bufa, bufb)
    sems = (sema, semb)

    pltpu.async_copy(wo2_hbm.at[idx_v.at[_CBLK]], bufa, sema)
    for blk in range(_CBLK):
        buf, sem = bufs[blk % 2], sems[blk % 2]
        pltpu.make_async_copy(wo2_hbm.at[idx_v.at[_CBLK + blk]], buf, sem).wait()
        if blk + 1 < _CBLK:
            pltpu.async_copy(wo2_hbm.at[idx_v.at[_CBLK + blk + 1]],
                             bufs[(blk + 1) % 2], sems[(blk + 1) % 2])

        @plsc.parallel_loop(0, 128, unroll=4)
        def dot_o(j, _blk=blk, _buf=buf):
            b = _blk * 128 + j
            jv = jnp.full((_L,), j, jnp.int32)
            pv = plsc.load_gather(pou_v, [jnp.full((_L,), b, jnp.int32)])
            offc = (pv & 1) << 6
            offo = (pv & 2) << 5
            bv = jnp.full((_L,), b, jnp.int32)
            d = jnp.zeros((_L,), jnp.float32)
            for q in range(_Q):
                cq = plsc.load_gather(c_all, [bv, offc + (iota + q * _L)])
                oq = plsc.load_gather(_buf, [jv, offo + (iota + q * _L)])
                d = d + cq * oq
            cum = plsc.cumsum(d)
            plsc.store_scatter(dblk, [jv], cum, mask=m15)

        accum_dblk()

    pltpu.async_copy(wo2_hbm.at[gn_v.at[0]], bufa, sema)

    def pair_body(i, carry):
        rr = i * 2
        for pb in range(2):
            r = rr + pb
            buf, sem = bufs[pb], sems[pb]
            pltpu.make_async_copy(wo2_hbm.at[gn_v.at[r]], buf, sem).wait()

            @pl.when(r + 1 < _NROWS)
            def _():
                pltpu.async_copy(wo2_hbm.at[gn_v.at[r + 1]],
                                 bufs[(pb + 1) % 2], sems[(pb + 1) % 2])

            @plsc.parallel_loop(0, 128, unroll=4)
            def dot_n(j, _buf=buf, _r=r):
                flat = _r * 128 + j
                jv = jnp.full((_L,), j, jnp.int32)
                pv = plsc.load_gather(pkn_v, [jnp.full((_L,), flat, jnp.int32)])
                bv = lax.shift_right_logical(pv, 2)
                offc = (pv & 1) << 6
                offx = (pv & 2) << 5
                d = jnp.zeros((_L,), jnp.float32)
                for q in range(_Q):
                    cq = plsc.load_gather(c_all, [bv, offc + (iota + q * _L)])
                    xq = plsc.load_gather(_buf, [jv, offx + (iota + q * _L)])
                    d = d + cq * xq
                cum = plsc.cumsum(d)
                plsc.store_scatter(dblk, [jv], -cum, mask=m15)

            accum_dblk()
        return carry

    lax.fori_loop(0, _NROWS // 2, pair_body, 0)

    pltpu.sync_copy(acc_v, out_hbm.at[pl.ds(wid * _L, _L)])


def _finish_body(p_ref, o_ref):
    o_ref[...] = (-jnp.sum(p_ref[...]) * (1.0 / B)).reshape(1, 1)


_finish = pl.pallas_call(
    _finish_body,
    out_shape=jax.ShapeDtypeStruct((1, 1), jnp.float32),
)


def kernel(center, outside, negative, W_center, W_outside):
    c_i = center.astype(jnp.int32).reshape(B)
    o_i = outside.astype(jnp.int32).reshape(B)
    n_i = negative.astype(jnp.int32).reshape(B * K)

    gc = (c_i >> 1).reshape(_NW, _CBLK, 128)
    go = (o_i >> 1).reshape(_NW, _CBLK, 128)
    gco = jnp.concatenate([gc, go], axis=1).reshape(_NW * 8, 128)
    gn = (n_i >> 1).reshape(B * K // 128, 128)

    cpar = c_i & 1
    bloc = (jnp.arange(B * K, dtype=jnp.int32) // K) % _BPW
    pkn = bloc * 4 + (n_i & 1) * 2 + jnp.repeat(cpar, K)
    pou = (o_i & 1) * 2 + cpar

    tails = jnp.stack([W_center[_NCH * _CH:].reshape(32, 128),
                       W_outside[_NCH * _CH:].reshape(32, 128)])
    wc2, wo2 = _sc_convert(W_center.T, W_outside.T, tails)
    partials = _sc_loss(gco, gn, pkn, pou, wc2, wo2)
    return _finish(partials.reshape(_NW, _L))[0, 0]


# speedup vs baseline: 1.5782x; 1.5782x over previous
import functools

import jax
import jax.numpy as jnp
from jax import lax
from jax.experimental import pallas as pl
from jax.experimental.pallas import tpu as pltpu
from jax.experimental.pallas import tpu_sc as plsc

VOC = 1000000
B = 16384
K = 20
EMB = 64

_NW = 32
_L = 16
_BPW = B // _NW
_CBLK = _BPW // 128
_NROWS = _BPW * K // 128
_Q = EMB // _L

_mesh = plsc.VectorSubcoreMesh(core_axis_name="c", subcore_axis_name="s")


@functools.partial(
    pl.kernel,
    out_type=jax.ShapeDtypeStruct((_NW * _L,), jnp.float32),
    mesh=_mesh,
    compiler_params=pltpu.CompilerParams(
        needs_layout_passes=False, use_tc_tiling_on_sc=False),
    scratch_types=[
        pltpu.VMEM((8, 128), jnp.int32),
        pltpu.VMEM((_NROWS, 128), jnp.int32),
        pltpu.VMEM((_BPW * K,), jnp.int32),
        pltpu.VMEM((_BPW, EMB), jnp.float32),
        pltpu.VMEM((128, EMB), jnp.float32),
        pltpu.VMEM((128, EMB), jnp.float32),
        pltpu.VMEM((128,), jnp.float32),
        pltpu.VMEM((_L,), jnp.float32),
        pltpu.SemaphoreType.DMA,
        pltpu.SemaphoreType.DMA,
    ],
)
def _sc_loss(gco_hbm, gn_hbm, pkn_hbm, wc_hbm, wo_hbm, out_hbm,
             idx_v, gn_v, pkn_v, c_all, bufa, bufb, dblk, acc_v,
             sema, semb):
    wid = lax.axis_index("s") * 2 + lax.axis_index("c")

    pltpu.sync_copy(gco_hbm.at[pl.ds(wid * 8, 8)], idx_v)
    pltpu.sync_copy(gn_hbm.at[pl.ds(wid * _NROWS, _NROWS)], gn_v)
    pltpu.sync_copy(pkn_hbm.at[pl.ds(wid * _BPW * K, _BPW * K)], pkn_v)

    iota = lax.iota(jnp.int32, _L)
    m15 = iota == _L - 1
    acc_v[...] = jnp.zeros((_L,), jnp.float32)

    cps = [
        pltpu.async_copy(wc_hbm.at[idx_v.at[blk]],
                         c_all.at[pl.ds(blk * 128, 128)], sema)
        for blk in range(_CBLK)
    ]
    for cp in cps:
        cp.wait()

    def accum_dblk():
        def rb(t, a):
            x = dblk[pl.ds(pl.multiple_of(t * _L, _L), _L)]
            e = jnp.exp(-jnp.abs(x))
            z = e / (e + 2.0)
            z2 = z * z
            l1p = z * (2.0 + z2 * (2.0 / 3.0 + z2 * (2.0 / 5.0 + z2 * (2.0 / 7.0))))
            return a + (jnp.minimum(x, 0.0) - l1p)

        acc_v[...] = lax.fori_loop(0, 128 // _L, rb, acc_v[...])

    bufs = (bufa, bufb)
    sems = (sema, semb)

    pltpu.async_copy(wo_hbm.at[idx_v.at[_CBLK]], bufa, sema)
    for blk in range(_CBLK):
        buf, sem = bufs[blk % 2], sems[blk % 2]
        pltpu.make_async_copy(wo_hbm.at[idx_v.at[_CBLK + blk]], buf, sem).wait()
        if blk + 1 < _CBLK:
            pltpu.async_copy(wo_hbm.at[idx_v.at[_CBLK + blk + 1]],
                             bufs[(blk + 1) % 2], sems[(blk + 1) % 2])

        @plsc.parallel_loop(0, 128, unroll=4)
        def dot_o(j, _blk=blk, _buf=buf):
            b = _blk * 128 + j
            jv = jnp.full((_L,), j, jnp.int32)
            d = jnp.zeros((_L,), jnp.float32)
            for q in range(_Q):
                d = d + (c_all[b, pl.ds(q * _L, _L)] * _buf[j, pl.ds(q * _L, _L)])
            cum = plsc.cumsum(d)
            plsc.store_scatter(dblk, [jv], cum, mask=m15)

        accum_dblk()

    pltpu.async_copy(wo_hbm.at[gn_v.at[0]], bufa, sema)

    def pair_body(i, carry):
        rr = i * 2
        for pb in range(2):
            r = rr + pb
            buf, sem = bufs[pb], sems[pb]
            pltpu.make_async_copy(wo_hbm.at[gn_v.at[r]], buf, sem).wait()

            @pl.when(r + 1 < _NROWS)
            def _():
                pltpu.async_copy(wo_hbm.at[gn_v.at[r + 1]],
                                 bufs[(pb + 1) % 2], sems[(pb + 1) % 2])

            @plsc.parallel_loop(0, 128, unroll=4)
            def dot_n(j, _buf=buf, _r=r):
                flat = _r * 128 + j
                jv = jnp.full((_L,), j, jnp.int32)
                bv = plsc.load_gather(pkn_v, [jnp.full((_L,), flat, jnp.int32)])
                d = jnp.zeros((_L,), jnp.float32)
                for q in range(_Q):
                    cq = plsc.load_gather(c_all, [bv, iota + q * _L])
                    d = d + cq * _buf[j, pl.ds(q * _L, _L)]
                cum = plsc.cumsum(d)
                plsc.store_scatter(dblk, [jv], -cum, mask=m15)

            accum_dblk()
        return carry

    lax.fori_loop(0, _NROWS // 2, pair_body, 0)

    pltpu.sync_copy(acc_v, out_hbm.at[pl.ds(wid * _L, _L)])


def _finish_body(p_ref, o_ref):
    o_ref[...] = (-jnp.sum(p_ref[...]) * (1.0 / B)).reshape(1, 1)


_finish = pl.pallas_call(
    _finish_body,
    out_shape=jax.ShapeDtypeStruct((1, 1), jnp.float32),
)


def kernel(center, outside, negative, W_center, W_outside):
    c_i = center.astype(jnp.int32).reshape(B)
    o_i = outside.astype(jnp.int32).reshape(B)
    n_i = negative.astype(jnp.int32).reshape(B * K)

    gc = c_i.reshape(_NW, _CBLK, 128)
    go = o_i.reshape(_NW, _CBLK, 128)
    gco = jnp.concatenate([gc, go], axis=1).reshape(_NW * 8, 128)
    gn = n_i.reshape(B * K // 128, 128)

    pkn = (jnp.arange(B * K, dtype=jnp.int32) // K) % _BPW

    partials = _sc_loss(gco, gn, pkn, W_center, W_outside)
    return _finish(partials.reshape(_NW, _L))[0, 0]
